# baseline (speedup 1.0000x reference)
import jax
import jax.numpy as jnp
from jax import lax
from jax.experimental import pallas as pl
from jax.experimental.pallas import tpu as pltpu
from jax.experimental.pallas import tpu_sc as plsc

_NC = 2
_NS = 16
_NW = _NC * _NS
_L = 16

_R = 8
_NBUF = 2


def _make_kernel(B, d):
    rows_per_w = B // _NW
    nchunk = rows_per_w // _R
    nsuper = nchunk // _NBUF
    nvec = d // _L

    mesh = plsc.VectorSubcoreMesh(
        core_axis_name="c", subcore_axis_name="s",
        num_cores=_NC, num_subcores=_NS,
    )

    scratch = (
        [pltpu.VMEM((d,), jnp.int32)] * 2
        + [pltpu.VMEM((_L,), jnp.int32)]
        + [pltpu.VMEM((_R, d), jnp.float32)] * (2 * _NBUF)
        + [pltpu.SemaphoreType.DMA] * (2 * _NBUF)
    )

    def body(x_hbm, perm_hbm, pos_hbm, out_hbm, perm_v, cp_v, pos_v,
             in0, in1, out0, out1, isem0, isem1, osem0, osem1):
        in_bufs = (in0, in1)
        out_bufs = (out0, out1)
        in_sems = (isem0, isem1)
        out_sems = (osem0, osem1)

        wid = lax.axis_index("s") * _NC + lax.axis_index("c")
        row0 = wid * rows_per_w

        pltpu.sync_copy(perm_hbm, perm_v)
        pltpu.sync_copy(pos_hbm, pos_v)

        pos = pos_v[...]
        pos = lax.rem(lax.rem(pos, d) + d, d)
        iota = lax.iota(jnp.int32, _L)

        @pl.loop(0, nvec)
        def _cp(j):
            idx = lax.rem(iota + (j * _L + pos), d)
            cp_v[pl.ds(j * _L, _L)] = plsc.load_gather(perm_v, [idx])

        def in_copy(c, b):
            return pltpu.make_async_copy(
                x_hbm.at[pl.ds(row0 + c * _R, _R), :], in_bufs[b], in_sems[b])

        def out_copy(c, b):
            return pltpu.make_async_copy(
                out_bufs[b], out_hbm.at[pl.ds(row0 + c * _R, _R), :],
                out_sems[b])

        for b in range(_NBUF):
            in_copy(b, b).start()

        @pl.loop(0, nsuper)
        def _super(g):
            c0 = g * _NBUF
            for b in range(_NBUF):
                c = c0 + b
                in_copy(c, b).wait()

                @pl.when(g > 0)
                def _():
                    out_copy(c, b).wait()

                @pl.loop(0, _R)
                def _row(r):
                    rsplat = jnp.full((_L,), 0, jnp.int32) + r
                    for j in range(nvec):
                        ci = cp_v[pl.ds(j * _L, _L)]
                        out_bufs[b][r, pl.ds(j * _L, _L)] = plsc.load_gather(
                            in_bufs[b], [rsplat, ci])

                out_copy(c, b).start()

                @pl.when(c + _NBUF < nchunk)
                def _():
                    in_copy(c + _NBUF, b).start()

        for b in range(_NBUF):
            out_copy(nchunk - _NBUF + b, b).wait()

    return pl.kernel(
        body,
        out_type=jax.ShapeDtypeStruct((B, d), jnp.float32),
        mesh=mesh,
        scratch_types=scratch,
        compiler_params=pltpu.CompilerParams(needs_layout_passes=False),
    )


def kernel(x, permutation, position):
    B, d = x.shape
    pos16 = jnp.broadcast_to(
        jnp.asarray(position, jnp.int32).reshape(()), (_L,))
    k = _make_kernel(B, d)
    return k(x, permutation.astype(jnp.int32), pos16)


# speedup vs baseline: 2.3598x; 2.3598x over previous
import jax
import jax.numpy as jnp
from jax import lax
from jax.experimental import pallas as pl
from jax.experimental.pallas import tpu as pltpu
from jax.experimental.pallas import tpu_sc as plsc

_NC = 2
_NS = 16
_NW = _NC * _NS
_L = 16

_R = 8
_NBUF = 2


def _make_kernel(B, d):
    rows_per_w = B // _NW
    nchunk = rows_per_w // _R
    nsuper = nchunk // _NBUF
    nvec = d // _L
    chunk_elems = _R * d

    mesh = plsc.VectorSubcoreMesh(
        core_axis_name="c", subcore_axis_name="s",
        num_cores=_NC, num_subcores=_NS,
    )

    scratch = (
        [pltpu.VMEM((d,), jnp.int32)] * 2
        + [pltpu.VMEM((_L,), jnp.int32)]
        + [pltpu.VMEM((chunk_elems,), jnp.float32)] * (2 * _NBUF)
        + [pltpu.SemaphoreType.DMA] * (2 * _NBUF)
    )

    def body(x_hbm, perm_hbm, pos_hbm, out_hbm, perm_v, cp_v, pos_v,
             in0, in1, out0, out1, isem0, isem1, osem0, osem1):
        in_bufs = (in0, in1)
        out_bufs = (out0, out1)
        in_sems = (isem0, isem1)
        out_sems = (osem0, osem1)

        wid = lax.axis_index("s") * _NC + lax.axis_index("c")
        elem0 = wid * (rows_per_w * d)

        pltpu.sync_copy(perm_hbm, perm_v)
        pltpu.sync_copy(pos_hbm, pos_v)

        pos = pos_v[...]
        pos = lax.rem(lax.rem(pos, d) + d, d)
        iota = lax.iota(jnp.int32, _L)

        @plsc.parallel_loop(0, nvec)
        def _cp(j):
            idx = lax.rem(iota + (j * _L + pos), d)
            cp_v[pl.ds(j * _L, _L)] = plsc.load_gather(perm_v, [idx])

        def in_copy(c, b):
            return pltpu.make_async_copy(
                x_hbm.at[pl.ds(elem0 + c * chunk_elems, chunk_elems)],
                in_bufs[b], in_sems[b])

        def out_copy(c, b):
            return pltpu.make_async_copy(
                out_bufs[b],
                out_hbm.at[pl.ds(elem0 + c * chunk_elems, chunk_elems)],
                out_sems[b])

        for b in range(_NBUF):
            in_copy(b, b).start()

        @pl.loop(0, nsuper)
        def _super(g):
            c0 = g * _NBUF
            for b in range(_NBUF):
                c = c0 + b
                in_copy(c, b).wait()

                @pl.when(g > 0)
                def _():
                    out_copy(c, b).wait()

                @plsc.parallel_loop(0, nvec)
                def _g(j):
                    ci = cp_v[pl.ds(j * _L, _L)]
                    o = j * _L
                    for r in range(_R):
                        out_bufs[b][pl.ds(r * d + o, _L)] = plsc.load_gather(
                            in_bufs[b], [ci + (r * d)])

                out_copy(c, b).start()

                @pl.when(c + _NBUF < nchunk)
                def _():
                    in_copy(c + _NBUF, b).start()

        for b in range(_NBUF):
            out_copy(nchunk - _NBUF + b, b).wait()

    return pl.kernel(
        body,
        out_type=jax.ShapeDtypeStruct((B * d,), jnp.float32),
        mesh=mesh,
        scratch_types=scratch,
        compiler_params=pltpu.CompilerParams(needs_layout_passes=False),
    )


def kernel(x, permutation, position):
    B, d = x.shape
    pos16 = jnp.broadcast_to(
        jnp.asarray(position, jnp.int32).reshape(()), (_L,))
    k = _make_kernel(B, d)
    out_flat = k(x.reshape(B * d), permutation.astype(jnp.int32), pos16)
    return out_flat.reshape(B, d)


# speedup vs baseline: 2.3651x; 1.0023x over previous
import jax
import jax.numpy as jnp
from jax import lax
from jax.experimental import pallas as pl
from jax.experimental.pallas import tpu as pltpu
from jax.experimental.pallas import tpu_sc as plsc

_NC = 2
_NS = 16
_NW = _NC * _NS
_L = 16

_R = 8
_NBUF = 2


def _make_kernel(B, d):
    rows_per_w = B // _NW
    nchunk = rows_per_w // _R
    nsuper = nchunk // _NBUF
    nvec = d // _L
    chunk_elems = _R * d

    mesh = plsc.VectorSubcoreMesh(
        core_axis_name="c", subcore_axis_name="s",
        num_cores=_NC, num_subcores=_NS,
    )

    scratch = (
        [pltpu.VMEM((d,), jnp.int32)] * 2
        + [pltpu.VMEM((_L,), jnp.int32)]
        + [pltpu.VMEM((chunk_elems,), jnp.float32)] * (2 * _NBUF)
        + [pltpu.SemaphoreType.DMA] * (2 * _NBUF)
    )

    def body(x_hbm, perm_hbm, pos_hbm, out_hbm, perm_v, cp_v, pos_v,
             in0, in1, out0, out1, isem0, isem1, osem0, osem1):
        in_bufs = (in0, in1)
        out_bufs = (out0, out1)
        in_sems = (isem0, isem1)
        out_sems = (osem0, osem1)

        wid = lax.axis_index("s") * _NC + lax.axis_index("c")
        elem0 = wid * (rows_per_w * d)

        pltpu.sync_copy(perm_hbm, perm_v)
        pltpu.sync_copy(pos_hbm, pos_v)

        pos = pos_v[...]
        pos = lax.rem(lax.rem(pos, d) + d, d)
        iota = lax.iota(jnp.int32, _L)

        @plsc.parallel_loop(0, nvec)
        def _cp(j):
            idx = lax.rem(iota + (j * _L + pos), d)
            cp_v[pl.ds(j * _L, _L)] = plsc.load_gather(perm_v, [idx])

        def in_copy(c, b):
            return pltpu.make_async_copy(
                x_hbm.at[pl.ds(elem0 + c * chunk_elems, chunk_elems)],
                in_bufs[b], in_sems[b])

        def out_copy(c, b):
            return pltpu.make_async_copy(
                out_bufs[b],
                out_hbm.at[pl.ds(elem0 + c * chunk_elems, chunk_elems)],
                out_sems[b])

        for b in range(_NBUF):
            in_copy(b, b).start()

        @pl.loop(0, nsuper)
        def _super(g):
            c0 = g * _NBUF
            for b in range(_NBUF):
                c = c0 + b
                in_copy(c, b).wait()

                @pl.when(g > 0)
                def _():
                    out_copy(c, b).wait()

                @plsc.parallel_loop(0, nvec, unroll=4)
                def _g(j):
                    ci = cp_v[pl.ds(j * _L, _L)]
                    o = j * _L
                    for r in range(_R):
                        out_bufs[b][pl.ds(r * d + o, _L)] = plsc.load_gather(
                            in_bufs[b], [ci + (r * d)])

                out_copy(c, b).start()

                @pl.when(c + _NBUF < nchunk)
                def _():
                    in_copy(c + _NBUF, b).start()

        for b in range(_NBUF):
            out_copy(nchunk - _NBUF + b, b).wait()

    return pl.kernel(
        body,
        out_type=jax.ShapeDtypeStruct((B * d,), jnp.float32),
        mesh=mesh,
        scratch_types=scratch,
        compiler_params=pltpu.CompilerParams(needs_layout_passes=False),
    )


def kernel(x, permutation, position):
    B, d = x.shape
    pos16 = jnp.broadcast_to(
        jnp.asarray(position, jnp.int32).reshape(()), (_L,))
    k = _make_kernel(B, d)
    out_flat = k(x.reshape(B * d), permutation.astype(jnp.int32), pos16)
    return out_flat.reshape(B, d)
